# Initial kernel scaffold; baseline (speedup 1.0000x reference)
import jax
import jax.numpy as jnp
from jax.experimental import pallas as pl


def kernel(features, geometry, neighbors, rel_mask, W):
    raise NotImplementedError("write your pallas kernel here")


# speedup vs baseline: 1.8323x; 1.8323x over previous
import functools

import jax
import jax.numpy as jnp
from jax import lax
from jax.experimental import pallas as pl
from jax.experimental.pallas import tpu as pltpu
from jax.experimental.pallas import tpu_sc as plsc

N = 10000
K = 32
CIN = 128
COUT = 128
NB_BASIS = 10
MAXR = 2.5
SIGMA = MAXR / NB_BASIS
INV2S2 = 1.0 / (2.0 * SIGMA * SIGMA)
E = N * K

_CH = 80
_L = 16


def _sc_gather_body(ft_hbm, xs_hbm, ys_hbm, zs_hbm, idx_hbm,
                    outf_hbm, outd_hbm,
                    idx_v, f_v, d2_v, xs_v, ys_v, zs_v, semf, nc):
    wid = lax.axis_index("s") * nc + lax.axis_index("c")
    ew = E // (nc * 16)
    nch = ew // _CH
    wbase = wid * ew

    pltpu.sync_copy(xs_hbm, xs_v)
    pltpu.sync_copy(ys_hbm, ys_v)
    pltpu.sync_copy(zs_hbm, zs_v)

    def body(c, carry):
        base = pl.multiple_of(wbase + c * _CH, 8)
        pltpu.sync_copy(idx_hbm.at[pl.ds(base, _CH)], idx_v)
        cf = pltpu.async_copy(ft_hbm.at[idx_v], f_v, semf)
        for g in range(_CH // _L):
            nbr = idx_v[pl.ds(g * _L, _L)]
            own = lax.shift_right_logical(
                base + g * _L + jnp.arange(_L, dtype=jnp.int32), 5)
            dx = plsc.load_gather(xs_v, [nbr]) - plsc.load_gather(xs_v, [own])
            dy = plsc.load_gather(ys_v, [nbr]) - plsc.load_gather(ys_v, [own])
            dz = plsc.load_gather(zs_v, [nbr]) - plsc.load_gather(zs_v, [own])
            d2_v[pl.ds(g * _L, _L)] = dx * dx + dy * dy + dz * dz
        cf.wait()
        pltpu.sync_copy(f_v, outf_hbm.at[pl.ds(base, _CH)])
        pltpu.sync_copy(d2_v, outd_hbm.at[pl.ds(base, _CH)])
        return carry

    lax.fori_loop(0, nch, body, 0)


def _sc_gather(ft, xs, ys, zs, idx):
    info = plsc.get_sparse_core_info()
    nc = info.num_cores
    mesh = plsc.VectorSubcoreMesh(core_axis_name="c", subcore_axis_name="s")
    fn = functools.partial(
        pl.kernel,
        mesh=mesh,
        out_type=(
            jax.ShapeDtypeStruct((E, CIN), jnp.float32),
            jax.ShapeDtypeStruct((E,), jnp.float32),
        ),
        scratch_types=[
            pltpu.VMEM((_CH,), jnp.int32),
            pltpu.VMEM((_CH, CIN), jnp.float32),
            pltpu.VMEM((_CH,), jnp.float32),
            pltpu.VMEM((N,), jnp.float32),
            pltpu.VMEM((N,), jnp.float32),
            pltpu.VMEM((N,), jnp.float32),
            pltpu.SemaphoreType.DMA,
        ],
        compiler_params=pltpu.CompilerParams(needs_layout_passes=False),
    )(functools.partial(_sc_gather_body, nc=nc))
    return fn(ft, xs, ys, zs, idx)


_NBLK = 200


def _tc_body(gf_ref, d2_ref, rm_ref, w_ref, out_ref):
    dist = jnp.sqrt(d2_ref[...] + 1e-12)
    rm = rm_ref[...]

    acc = jnp.zeros((_NBLK, COUT), jnp.float32)
    for b in range(NB_BASIS):
        c_b = b * (MAXR / (NB_BASIS - 1))
        rb = jnp.exp((dist - c_b) ** 2 * (-INV2S2)) * rm
        t = jnp.zeros((_NBLK, CIN), jnp.float32)
        for k in range(K):
            t = t + rb[:, k:k + 1] * gf_ref[:, k, :]
        acc = acc + lax.dot_general(
            t, w_ref[b],
            dimension_numbers=(((1,), (1,)), ((), ())),
            preferred_element_type=jnp.float32)
    out_ref[...] = acc


def _tc_compute(gf3, d2, rel_mask, w):
    grid = (N // _NBLK,)
    return pl.pallas_call(
        _tc_body,
        grid=grid,
        in_specs=[
            pl.BlockSpec((_NBLK, K, CIN), lambda i: (i, 0, 0)),
            pl.BlockSpec((_NBLK, K), lambda i: (i, 0)),
            pl.BlockSpec((_NBLK, K), lambda i: (i, 0)),
            pl.BlockSpec((NB_BASIS, COUT, CIN), lambda i: (0, 0, 0)),
        ],
        out_specs=pl.BlockSpec((_NBLK, COUT), lambda i: (i, 0)),
        out_shape=jax.ShapeDtypeStruct((N, COUT), jnp.float32),
        compiler_params=pltpu.CompilerParams(
            dimension_semantics=("arbitrary",)),
    )(gf3, d2, rel_mask, w)


def kernel(features, geometry, neighbors, rel_mask, W):
    ft = features.T
    xs = geometry[:, 0]
    ys = geometry[:, 1]
    zs = geometry[:, 2]
    idx = neighbors.reshape(-1).astype(jnp.int32)
    gf, d2 = _sc_gather(ft, xs, ys, zs, idx)
    outT = _tc_compute(gf.reshape(N, K, CIN), d2.reshape(N, K), rel_mask, W)
    return outT.T
